# baseline (speedup 1.0000x reference)
import functools
import jax
import jax.numpy as jnp
from jax import lax
from jax.experimental import pallas as pl
from jax.experimental.pallas import tpu as pltpu
from jax.experimental.pallas import tpu_sc as plsc

NUM_CHARS = 256
EMB_DIM = 16
BATCH = 16384
MSG_LEN = 256
TOTAL = BATCH * MSG_LEN

NC = 2
NS = 16
NW = NC * NS
L = 16

CHUNK = 2048
IDX_PER_W = TOTAL // NW
STEPS = IDX_PER_W // CHUNK
NVEC = CHUNK // L
UNROLL = 4


def _sc_body(x_hbm, tabT_hbm, out_hbm,
             tab_v, idx_a, idx_b, rows_a, rows_b, sem_i, sem_oa, sem_ob):
    wid = lax.axis_index("s") * NC + lax.axis_index("c")
    base0 = wid * IDX_PER_W

    pltpu.sync_copy(tabT_hbm, tab_v)

    iota = jnp.arange(L, dtype=jnp.int32)
    jsp = [jnp.full((L,), j, dtype=jnp.int32) for j in range(EMB_DIM)]

    bufs = ((idx_a, rows_a, sem_oa), (idx_b, rows_b, sem_ob))

    def phase(i, parity):
        idx_v, rows_v, sem_o = bufs[parity]
        idx_n = bufs[1 - parity][0]
        base = base0 + i * CHUNK

        @pl.when(i == 0)
        def _():
            pltpu.sync_copy(x_hbm.at[pl.ds(base, CHUNK)], idx_v)

        @pl.when(i > 0)
        def _():
            pltpu.make_async_copy(
                x_hbm.at[pl.ds(base, CHUNK)], idx_v, sem_i).wait()

        @pl.when(i + 1 < STEPS)
        def _():
            pltpu.async_copy(
                x_hbm.at[pl.ds(base + CHUNK, CHUNK)], idx_n, sem_i)


        def group(g):
            idx_vec = idx_v[pl.ds(g * L, L)]
            row_vec = g * L + iota
            for j in range(EMB_DIM):
                vals = plsc.load_gather(tab_v, [jsp[j], idx_vec])
                plsc.store_scatter(rows_v, [row_vec, jsp[j]], vals)

        def body(t, _):
            for u in range(UNROLL):
                group(t * UNROLL + u)
            return ()

        lax.fori_loop(0, NVEC // UNROLL, body, ())


    def step(t, _):
        phase(2 * t, 0)
        phase(2 * t + 1, 1)
        return ()

    lax.fori_loop(0, STEPS // 2, step, ())

    last = base0 + (STEPS - 2) * CHUNK
    pltpu.async_copy(rows_a, out_hbm.at[pl.ds(last, CHUNK)], sem_oa).wait()
    pltpu.async_copy(
        rows_b, out_hbm.at[pl.ds(last + CHUNK, CHUNK)], sem_ob).wait()


@jax.jit
def _encode(x1d, tabT):
    mesh = plsc.VectorSubcoreMesh(core_axis_name="c", subcore_axis_name="s")
    run = pl.kernel(
        _sc_body,
        out_type=jax.ShapeDtypeStruct((TOTAL, EMB_DIM), jnp.float32),
        mesh=mesh,
        scratch_types=[
            pltpu.VMEM((EMB_DIM, NUM_CHARS), jnp.float32),
            pltpu.VMEM((CHUNK,), jnp.int32),
            pltpu.VMEM((CHUNK,), jnp.int32),
            pltpu.VMEM((CHUNK, EMB_DIM), jnp.float32),
            pltpu.VMEM((CHUNK, EMB_DIM), jnp.float32),
            pltpu.SemaphoreType.DMA,
            pltpu.SemaphoreType.DMA,
            pltpu.SemaphoreType.DMA,
        ],
        compiler_params=pltpu.CompilerParams(
            use_tc_tiling_on_sc=False, needs_layout_passes=False),
    )
    return run(x1d, tabT)


def kernel(x, emb_table):
    x1d = jnp.asarray(x, jnp.int32).reshape(TOTAL)
    tabT = emb_table.T.reshape(EMB_DIM, NUM_CHARS)
    y = _encode(x1d, tabT)
    return y.reshape(BATCH, MSG_LEN * EMB_DIM)


# speedup vs baseline: 2.8417x; 2.8417x over previous
import functools
import jax
import jax.numpy as jnp
from jax import lax
from jax.experimental import pallas as pl
from jax.experimental.pallas import tpu as pltpu
from jax.experimental.pallas import tpu_sc as plsc

NUM_CHARS = 256
EMB_DIM = 16
BATCH = 16384
MSG_LEN = 256
TOTAL = BATCH * MSG_LEN

NC = 2
NS = 16
NW = NC * NS
L = 16

CHUNK = 2048
IDX_PER_W = TOTAL // NW
STEPS = IDX_PER_W // CHUNK
NVEC = CHUNK // L
UNROLL = 4


def _sc_body(x_hbm, tabT_hbm, out_hbm,
             tab_v, idx_a, idx_b, rows_a, rows_b, sem_i, sem_oa, sem_ob):
    wid = lax.axis_index("s") * NC + lax.axis_index("c")
    base0 = wid * IDX_PER_W

    pltpu.sync_copy(tabT_hbm, tab_v)

    iota = jnp.arange(L, dtype=jnp.int32)
    jsp = [jnp.full((L,), j, dtype=jnp.int32) for j in range(EMB_DIM)]

    bufs = ((idx_a, rows_a, sem_oa), (idx_b, rows_b, sem_ob))

    def phase(i, parity):
        idx_v, rows_v, sem_o = bufs[parity]
        idx_n = bufs[1 - parity][0]
        base = base0 + i * CHUNK

        @pl.when(i == 0)
        def _():
            pltpu.sync_copy(x_hbm.at[pl.ds(base, CHUNK)], idx_v)

        @pl.when(i > 0)
        def _():
            pltpu.make_async_copy(
                x_hbm.at[pl.ds(base, CHUNK)], idx_v, sem_i).wait()

        @pl.when(i + 1 < STEPS)
        def _():
            pltpu.async_copy(
                x_hbm.at[pl.ds(base + CHUNK, CHUNK)], idx_n, sem_i)

        @pl.when(i >= 2)
        def _():
            pltpu.make_async_copy(
                rows_v, out_hbm.at[pl.ds(base, CHUNK)], sem_o).wait()

        def group(g):
            idx_vec = idx_v[pl.ds(g * L, L)]
            row_vec = g * L + iota
            for j in range(EMB_DIM):
                vals = plsc.load_gather(tab_v, [jsp[j], idx_vec])
                plsc.store_scatter(rows_v, [row_vec, jsp[j]], vals)

        @plsc.parallel_loop(0, NVEC, 1, unroll=UNROLL)
        def _(g):
            group(g)

        pltpu.async_copy(rows_v, out_hbm.at[pl.ds(base, CHUNK)], sem_o)

    def step(t, _):
        phase(2 * t, 0)
        phase(2 * t + 1, 1)
        return ()

    lax.fori_loop(0, STEPS // 2, step, ())

    last = base0 + (STEPS - 2) * CHUNK
    pltpu.make_async_copy(rows_a, out_hbm.at[pl.ds(last, CHUNK)], sem_oa).wait()
    pltpu.make_async_copy(
        rows_b, out_hbm.at[pl.ds(last + CHUNK, CHUNK)], sem_ob).wait()


@jax.jit
def _encode(x1d, tabT):
    mesh = plsc.VectorSubcoreMesh(core_axis_name="c", subcore_axis_name="s")
    run = pl.kernel(
        _sc_body,
        out_type=jax.ShapeDtypeStruct((TOTAL, EMB_DIM), jnp.float32),
        mesh=mesh,
        scratch_types=[
            pltpu.VMEM((EMB_DIM, NUM_CHARS), jnp.float32),
            pltpu.VMEM((CHUNK,), jnp.int32),
            pltpu.VMEM((CHUNK,), jnp.int32),
            pltpu.VMEM((CHUNK, EMB_DIM), jnp.float32),
            pltpu.VMEM((CHUNK, EMB_DIM), jnp.float32),
            pltpu.SemaphoreType.DMA,
            pltpu.SemaphoreType.DMA,
            pltpu.SemaphoreType.DMA,
        ],
        compiler_params=pltpu.CompilerParams(
            use_tc_tiling_on_sc=False, needs_layout_passes=False),
    )
    return run(x1d, tabT)


def kernel(x, emb_table):
    x1d = jnp.asarray(x, jnp.int32).reshape(TOTAL)
    tabT = emb_table.T.reshape(EMB_DIM, NUM_CHARS)
    y = _encode(x1d, tabT)
    return y.reshape(BATCH, MSG_LEN * EMB_DIM)
